# baseline (speedup 1.0000x reference)
import functools

import jax
import jax.numpy as jnp
import numpy as np
from jax import lax
from jax.experimental import pallas as pl
from jax.experimental.pallas import tpu as pltpu
from jax.experimental.pallas import tpu_sc as plsc

U = 32
ED = 16
N_NODES = 10000
N_EDGES = 160000

CHUNK = 128
EP = 163840
NROWS = EP // CHUNK
NPAD = 10048
STRIPE = NPAD // 16
DUMMY = N_NODES

BE = 1024
HIGH = lax.Precision.DEFAULT
UNROLL = 8


def _sc_mesh():
    return plsc.VectorSubcoreMesh(core_axis_name="core", subcore_axis_name="subcore")


SC_PARAMS = pltpu.CompilerParams(use_tc_tiling_on_sc=False)


def _gather(node_features, src_rows):
    @functools.partial(
        pl.kernel,
        out_type=jax.ShapeDtypeStruct((EP, U), jnp.float32),
        mesh=_sc_mesh(),
        compiler_params=SC_PARAMS,
    )
    def gather_kernel(table_hbm, idx_hbm, out_hbm):
        def body(i_vmem, o_vmem):
            pltpu.sync_copy(table_hbm.at[i_vmem.at[0]], o_vmem)

        pltpu.emit_pipeline(
            body,
            grid=(NROWS,),
            in_specs=[pl.BlockSpec((1, CHUNK), lambda i: (0, i))],
            out_specs=[pl.BlockSpec((CHUNK, U), lambda i: (i, 0))],
            core_axis_name=("core", "subcore"),
            dimension_semantics=(pltpu.PARALLEL,),
        )(idx_hbm, out_hbm)

    return gather_kernel(node_features, src_rows)


def _dense_body(ef_ref, src_ref, w1_ref, b1_ref, w2_ref, b2_ref, sel_ref, out_ref):
    h = jnp.maximum(
        jnp.dot(ef_ref[...], w1_ref[...], precision=HIGH) + b1_ref[...], 0.0
    )
    full = jnp.dot(h, w2_ref[...], precision=HIGH) + b2_ref[...]
    src = src_ref[...]
    src_t = jnp.concatenate([src] * U, axis=1)
    out_ref[...] = jnp.dot(full * src_t, sel_ref[...], precision=HIGH)


def _dense(ef_p, src_feat, W1, b1, W2, b2, sel):
    grid = EP // BE
    return pl.pallas_call(
        _dense_body,
        grid=(grid,),
        in_specs=[
            pl.BlockSpec((BE, ED), lambda i: (i, 0)),
            pl.BlockSpec((BE, U), lambda i: (i, 0)),
            pl.BlockSpec((ED, U), lambda i: (0, 0)),
            pl.BlockSpec((1, U), lambda i: (0, 0)),
            pl.BlockSpec((U, U * U), lambda i: (0, 0)),
            pl.BlockSpec((1, U * U), lambda i: (0, 0)),
            pl.BlockSpec((U * U, U), lambda i: (0, 0)),
        ],
        out_specs=pl.BlockSpec((BE, U), lambda i: (i, 0)),
        out_shape=jax.ShapeDtypeStruct((EP, U), jnp.float32),
    )(ef_p, src_feat, W1, b1, W2, b2, sel)


def _scatter(messages, dst_rows, zeros32, zeros16, ones16):
    @functools.partial(
        pl.kernel,
        out_type=[
            jax.ShapeDtypeStruct((2, NPAD, U), jnp.float32),
            jax.ShapeDtypeStruct((2, NPAD, 16), jnp.float32),
        ],
        mesh=_sc_mesh(),
        compiler_params=SC_PARAMS,
        scratch_types=[
            pltpu.VMEM_SHARED((NPAD, U), jnp.float32),
            pltpu.VMEM_SHARED((NPAD, 16), jnp.float32),
            pltpu.VMEM((CHUNK, 16), jnp.float32),
        ],
    )
    def scatter_kernel(msg_hbm, idx_hbm, z32_hbm, z16_hbm, ones_hbm,
                       sums_out, cnt_out, sums_sh, cnt_sh, ones_v):
        cid = lax.axis_index("core")
        sid = lax.axis_index("subcore")
        base = sid * STRIPE
        pltpu.sync_copy(z32_hbm, sums_sh.at[pl.ds(base, STRIPE)])
        pltpu.sync_copy(z16_hbm, cnt_sh.at[pl.ds(base, STRIPE)])
        pltpu.sync_copy(ones_hbm, ones_v)
        plsc.subcore_barrier()

        def body(m_vmem, i_vmem):
            pltpu.sync_copy(m_vmem, sums_sh.at[i_vmem.at[0]], add=True)
            pltpu.sync_copy(ones_v, cnt_sh.at[i_vmem.at[0]], add=True)

        pltpu.emit_pipeline(
            body,
            grid=(NROWS,),
            in_specs=[
                pl.BlockSpec((CHUNK, U), lambda i: (i, 0)),
                pl.BlockSpec((1, CHUNK), lambda i: (0, i)),
            ],
            out_specs=[],
            core_axis_name=("core", "subcore"),
            dimension_semantics=(pltpu.PARALLEL,),
        )(msg_hbm, idx_hbm)
        plsc.subcore_barrier()
        pltpu.sync_copy(sums_sh.at[pl.ds(base, STRIPE)],
                        sums_out.at[cid, pl.ds(base, STRIPE)])
        pltpu.sync_copy(cnt_sh.at[pl.ds(base, STRIPE)],
                        cnt_out.at[cid, pl.ds(base, STRIPE)])

    return scatter_kernel(messages, dst_rows, zeros32, zeros16, ones16)


def _gru_body(sums_ref, cnt_ref, gk_ref, grk_ref, gb_ref, h0_ref, out_ref, xm_ref):
    sums = sums_ref[0, 0:N_NODES, :] + sums_ref[1, 0:N_NODES, :]
    cnt = cnt_ref[0, 0:N_NODES, 0:1] + cnt_ref[1, 0:N_NODES, 0:1]
    agg = sums / jnp.maximum(cnt, 1.0)
    xm_ref[...] = jnp.dot(agg, gk_ref[...], precision=HIGH) + gb_ref[0:1, :]

    grk = grk_ref[...]
    gb1 = gb_ref[1:2, :]

    def step(xm, h):
        hm = jnp.sum(jnp.transpose(h) * grk, axis=0, keepdims=True) + gb1
        a = xm + hm
        s = 0.5 * jnp.tanh(0.5 * a[:, 0:2 * U]) + 0.5
        z = s[:, 0:U]
        r = s[:, U:2 * U]
        hh = jnp.tanh(xm[:, 2 * U:3 * U] + r * hm[:, 2 * U:3 * U])
        return hh + z * (h - hh)

    def block(b, h):
        xm8 = xm_ref[pl.ds(b * UNROLL, UNROLL), :]
        for k in range(UNROLL):
            h = step(xm8[k:k + 1, :], h)
        return h

    out_ref[...] = lax.fori_loop(0, N_NODES // UNROLL, block, h0_ref[...])


def _gru(sums_p, cnt_p, gru_kernel, gru_rkernel, gru_bias, hidden_state):
    return pl.pallas_call(
        _gru_body,
        grid=(1,),
        in_specs=[
            pl.BlockSpec((2, NPAD, U), lambda i: (0, 0, 0)),
            pl.BlockSpec((2, NPAD, 16), lambda i: (0, 0, 0)),
            pl.BlockSpec((U, 3 * U), lambda i: (0, 0)),
            pl.BlockSpec((U, 3 * U), lambda i: (0, 0)),
            pl.BlockSpec((2, 3 * U), lambda i: (0, 0)),
            pl.BlockSpec((1, U), lambda i: (0, 0)),
        ],
        out_specs=pl.BlockSpec((1, U), lambda i: (0, 0)),
        out_shape=jax.ShapeDtypeStruct((1, U), jnp.float32),
        scratch_shapes=[pltpu.VMEM((N_NODES, 3 * U), jnp.float32)],
    )(sums_p, cnt_p, gru_kernel, gru_rkernel, gru_bias, hidden_state)


def kernel(node_features, edge_features, edge_indices, hidden_state,
           W1, b1, W2, b2, gru_kernel, gru_rkernel, gru_bias):
    pad = EP - N_EDGES
    src_idx = jnp.concatenate(
        [edge_indices[0], jnp.zeros((pad,), jnp.int32)]).reshape(1, EP)
    dst_idx = jnp.concatenate(
        [edge_indices[1], jnp.full((pad,), DUMMY, jnp.int32)]).reshape(1, EP)
    ef_p = jnp.concatenate(
        [edge_features, jnp.zeros((pad, ED), jnp.float32)], axis=0)

    sel = jnp.asarray(np.repeat(np.eye(U, dtype=np.float32), U, axis=0))
    zeros32 = jnp.zeros((STRIPE, U), jnp.float32)
    zeros16 = jnp.zeros((STRIPE, 16), jnp.float32)
    ones16 = jnp.ones((CHUNK, 16), jnp.float32)

    src_feat = _gather(node_features, src_idx)
    messages = _dense(ef_p, src_feat, W1, b1.reshape(1, U), W2,
                      b2.reshape(1, U * U), sel)
    sums_p, cnt_p = _scatter(messages, dst_idx, zeros32, zeros16, ones16)
    new_state = _gru(sums_p, cnt_p, gru_kernel, gru_rkernel, gru_bias,
                     hidden_state)
    return new_state[0], new_state


# speedup vs baseline: 2.8853x; 2.8853x over previous
import functools

import jax
import jax.numpy as jnp
import numpy as np
from jax import lax
from jax.experimental import pallas as pl
from jax.experimental.pallas import tpu as pltpu
from jax.experimental.pallas import tpu_sc as plsc

U = 32
ED = 16
N_NODES = 10000
N_EDGES = 160000

CHUNK = 128
EP = 163840
NROWS = EP // CHUNK
NPAD = 10048
STRIPE = NPAD // 16
DUMMY = N_NODES

BE = 1024
HIGH = lax.Precision.DEFAULT
UNROLL = 8


def _sc_mesh():
    return plsc.VectorSubcoreMesh(core_axis_name="core", subcore_axis_name="subcore")


SC_PARAMS = pltpu.CompilerParams(use_tc_tiling_on_sc=False)


def _gather(node_features, src_rows):
    @functools.partial(
        pl.kernel,
        out_type=jax.ShapeDtypeStruct((EP, U), jnp.float32),
        mesh=_sc_mesh(),
        compiler_params=SC_PARAMS,
    )
    def gather_kernel(table_hbm, idx_hbm, out_hbm):
        def body(i_vmem, o_vmem):
            pltpu.sync_copy(table_hbm.at[i_vmem.at[0]], o_vmem)

        pltpu.emit_pipeline(
            body,
            grid=(NROWS,),
            in_specs=[pl.BlockSpec((1, CHUNK), lambda i: (0, i))],
            out_specs=[pl.BlockSpec((CHUNK, U), lambda i: (i, 0))],
            core_axis_name=("core", "subcore"),
            dimension_semantics=(pltpu.PARALLEL,),
        )(idx_hbm, out_hbm)

    return gather_kernel(node_features, src_rows)


def _dense_body(ef_ref, src_ref, w1_ref, b1_ref, w2_ref, b2_ref, sel_ref, out_ref):
    h = jnp.maximum(
        jnp.dot(ef_ref[...], w1_ref[...], precision=HIGH) + b1_ref[...], 0.0
    )
    full = jnp.dot(h, w2_ref[...], precision=HIGH) + b2_ref[...]
    src = src_ref[...]
    src_t = jnp.concatenate([src] * U, axis=1)
    out_ref[...] = jnp.dot(full * src_t, sel_ref[...], precision=HIGH)


def _dense(ef_p, src_feat, W1, b1, W2, b2, sel):
    grid = EP // BE
    return pl.pallas_call(
        _dense_body,
        grid=(grid,),
        in_specs=[
            pl.BlockSpec((BE, ED), lambda i: (i, 0)),
            pl.BlockSpec((BE, U), lambda i: (i, 0)),
            pl.BlockSpec((ED, U), lambda i: (0, 0)),
            pl.BlockSpec((1, U), lambda i: (0, 0)),
            pl.BlockSpec((U, U * U), lambda i: (0, 0)),
            pl.BlockSpec((1, U * U), lambda i: (0, 0)),
            pl.BlockSpec((U * U, U), lambda i: (0, 0)),
        ],
        out_specs=pl.BlockSpec((BE, U), lambda i: (i, 0)),
        out_shape=jax.ShapeDtypeStruct((EP, U), jnp.float32),
    )(ef_p, src_feat, W1, b1, W2, b2, sel)


def _scatter(messages, dst_rows, zeros32, zeros16, ones16):
    @functools.partial(
        pl.kernel,
        out_type=[
            jax.ShapeDtypeStruct((2, NPAD, U), jnp.float32),
            jax.ShapeDtypeStruct((2, NPAD, 16), jnp.float32),
        ],
        mesh=_sc_mesh(),
        compiler_params=SC_PARAMS,
        scratch_types=[
            pltpu.VMEM_SHARED((NPAD, U), jnp.float32),
            pltpu.VMEM_SHARED((NPAD, 16), jnp.float32),
            pltpu.VMEM((CHUNK, 16), jnp.float32),
        ],
    )
    def scatter_kernel(msg_hbm, idx_hbm, z32_hbm, z16_hbm, ones_hbm,
                       sums_out, cnt_out, sums_sh, cnt_sh, ones_v):
        cid = lax.axis_index("core")
        sid = lax.axis_index("subcore")
        base = sid * STRIPE
        pltpu.sync_copy(z32_hbm, sums_sh.at[pl.ds(base, STRIPE)])
        pltpu.sync_copy(z16_hbm, cnt_sh.at[pl.ds(base, STRIPE)])
        pltpu.sync_copy(ones_hbm, ones_v)
        plsc.subcore_barrier()

        def body(m_vmem, i_vmem):
            pltpu.sync_copy(m_vmem, sums_sh.at[i_vmem.at[0]], add=True)
            pltpu.sync_copy(ones_v, cnt_sh.at[i_vmem.at[0]], add=True)

        pltpu.emit_pipeline(
            body,
            grid=(NROWS,),
            in_specs=[
                pl.BlockSpec((CHUNK, U), lambda i: (i, 0)),
                pl.BlockSpec((1, CHUNK), lambda i: (0, i)),
            ],
            out_specs=[],
            core_axis_name=("core", "subcore"),
            dimension_semantics=(pltpu.PARALLEL,),
        )(msg_hbm, idx_hbm)
        plsc.subcore_barrier()
        pltpu.sync_copy(sums_sh.at[pl.ds(base, STRIPE)],
                        sums_out.at[cid, pl.ds(base, STRIPE)])
        pltpu.sync_copy(cnt_sh.at[pl.ds(base, STRIPE)],
                        cnt_out.at[cid, pl.ds(base, STRIPE)])

    return scatter_kernel(messages, dst_rows, zeros32, zeros16, ones16)


def _gru_body(sums_ref, cnt_ref, kz_ref, kr_ref, kh_ref,
              rz_ref, rr_ref, rh_ref, bxz_ref, bxr_ref, bxh_ref, bhr_ref,
              h0_ref, out_ref, xz_ref, xr_ref, xh_ref):
    sums = sums_ref[0, 0:N_NODES, :] + sums_ref[1, 0:N_NODES, :]
    cnt = cnt_ref[0, 0:N_NODES, 0:1] + cnt_ref[1, 0:N_NODES, 0:1]
    agg = sums / jnp.maximum(cnt, 1.0)
    xz_ref[...] = jnp.dot(agg, kz_ref[...]) + bxz_ref[...]
    xr_ref[...] = jnp.dot(agg, kr_ref[...]) + bxr_ref[...]
    xh_ref[...] = jnp.dot(agg, kh_ref[...]) + bxh_ref[...]

    bhr = bhr_ref[...]
    rz16 = rz_ref[...]
    rr16 = rr_ref[...]
    rh16 = rh_ref[...]

    def step(t, h):
        xz = xz_ref[pl.ds(t, 1), :]
        xr = xr_ref[pl.ds(t, 1), :]
        xh = xh_ref[pl.ds(t, 1), :]
        hb = h.astype(jnp.bfloat16)
        dz = jnp.dot(hb, rz16, preferred_element_type=jnp.float32)
        dr = jnp.dot(hb, rr16, preferred_element_type=jnp.float32)
        dh = jnp.dot(hb, rh16, preferred_element_type=jnp.float32)
        tz = jnp.tanh(xz + dz)
        tr = jnp.tanh(xr + dr)
        hh = jnp.tanh(xh + (tr + 1.0) * (dh + bhr))
        return 0.5 * (h + hh + tz * (h - hh))

    def block(b, h):
        for k in range(UNROLL):
            h = step(b * UNROLL + k, h)
        return h

    out_ref[...] = lax.fori_loop(0, N_NODES // UNROLL, block, h0_ref[...])


def _gru(sums_p, cnt_p, gru_kernel, gru_rkernel, gru_bias, hidden_state):
    kz = gru_kernel[:, 0:U] * 0.5
    kr = gru_kernel[:, U:2 * U] * 0.5
    kh = gru_kernel[:, 2 * U:3 * U]
    rz = (gru_rkernel[:, 0:U] * 0.5).astype(jnp.bfloat16)
    rr = (gru_rkernel[:, U:2 * U] * 0.5).astype(jnp.bfloat16)
    rh = (gru_rkernel[:, 2 * U:3 * U] * 0.5).astype(jnp.bfloat16)
    bxz = ((gru_bias[0, 0:U] + gru_bias[1, 0:U]) * 0.5).reshape(1, U)
    bxr = ((gru_bias[0, U:2 * U] + gru_bias[1, U:2 * U]) * 0.5).reshape(1, U)
    bxh = gru_bias[0, 2 * U:3 * U].reshape(1, U)
    bhr = (gru_bias[1, 2 * U:3 * U] * 0.5).reshape(1, U)

    full = lambda s: pl.BlockSpec(s, lambda i: tuple(0 for _ in s))
    return pl.pallas_call(
        _gru_body,
        grid=(1,),
        in_specs=[
            full((2, NPAD, U)),
            full((2, NPAD, 16)),
            full((U, U)), full((U, U)), full((U, U)),
            full((U, U)), full((U, U)), full((U, U)),
            full((1, U)), full((1, U)), full((1, U)), full((1, U)),
            full((1, U)),
        ],
        out_specs=pl.BlockSpec((1, U), lambda i: (0, 0)),
        out_shape=jax.ShapeDtypeStruct((1, U), jnp.float32),
        scratch_shapes=[pltpu.VMEM((N_NODES, U), jnp.float32)] * 3,
    )(sums_p, cnt_p, kz, kr, kh, rz, rr, rh, bxz, bxr, bxh, bhr, hidden_state)


def kernel(node_features, edge_features, edge_indices, hidden_state,
           W1, b1, W2, b2, gru_kernel, gru_rkernel, gru_bias):
    pad = EP - N_EDGES
    src_idx = jnp.concatenate(
        [edge_indices[0], jnp.zeros((pad,), jnp.int32)]).reshape(1, EP)
    dst_idx = jnp.concatenate(
        [edge_indices[1], jnp.full((pad,), DUMMY, jnp.int32)]).reshape(1, EP)
    ef_p = jnp.concatenate(
        [edge_features, jnp.zeros((pad, ED), jnp.float32)], axis=0)

    sel = jnp.asarray(np.repeat(np.eye(U, dtype=np.float32), U, axis=0))
    zeros32 = jnp.zeros((STRIPE, U), jnp.float32)
    zeros16 = jnp.zeros((STRIPE, 16), jnp.float32)
    ones16 = jnp.ones((CHUNK, 16), jnp.float32)

    src_feat = _gather(node_features, src_idx)
    messages = _dense(ef_p, src_feat, W1, b1.reshape(1, U), W2,
                      b2.reshape(1, U * U), sel)
    sums_p, cnt_p = _scatter(messages, dst_idx, zeros32, zeros16, ones16)
    new_state = _gru(sums_p, cnt_p, gru_kernel, gru_rkernel, gru_bias,
                     hidden_state)
    return new_state[0], new_state


# speedup vs baseline: 2.9945x; 1.0378x over previous
import functools

import jax
import jax.numpy as jnp
import numpy as np
from jax import lax
from jax.experimental import pallas as pl
from jax.experimental.pallas import tpu as pltpu
from jax.experimental.pallas import tpu_sc as plsc

U = 32
ED = 16
N_NODES = 10000
N_EDGES = 160000

CHUNK = 128
EP = 163840
NROWS = EP // CHUNK
NPAD = 10048
STRIPE = NPAD // 16
DUMMY = N_NODES

BE = 2048
HIGH = lax.Precision.DEFAULT
UNROLL = 8


def _sc_mesh():
    return plsc.VectorSubcoreMesh(core_axis_name="core", subcore_axis_name="subcore")


SC_PARAMS = pltpu.CompilerParams(use_tc_tiling_on_sc=False)


GK = 8
NG = NROWS // GK


def _gather(node_features, src_rows, n_rows):
    @functools.partial(
        pl.kernel,
        out_type=jax.ShapeDtypeStruct((n_rows * CHUNK, U), jnp.float32),
        mesh=_sc_mesh(),
        compiler_params=SC_PARAMS,
    )
    def gather_kernel(table_hbm, idx_hbm, out_hbm):
        def body(i_vmem, o_vmem):
            pltpu.sync_copy(table_hbm.at[i_vmem.at[0]], o_vmem)

        pltpu.emit_pipeline(
            body,
            grid=(n_rows,),
            in_specs=[pl.BlockSpec((1, CHUNK), lambda i: (0, i))],
            out_specs=[pl.BlockSpec((CHUNK, U), lambda i: (i, 0))],
            core_axis_name=("core", "subcore"),
            dimension_semantics=(pltpu.PARALLEL,),
        )(idx_hbm, out_hbm)

    return gather_kernel(node_features, src_rows)


def _dense_body(ef_ref, src_ref, w1_ref, b1_ref, w2_ref, b2_ref, sel_ref, out_ref):
    h = jnp.maximum(
        jnp.dot(ef_ref[...], w1_ref[...], precision=HIGH) + b1_ref[...], 0.0
    )
    full = jnp.dot(h, w2_ref[...], precision=HIGH) + b2_ref[...]
    src = src_ref[...]
    src_t = jnp.concatenate([src] * U, axis=1)
    out_ref[...] = jnp.dot(full * src_t, sel_ref[...], precision=HIGH)


def _dense(ef_p, src_feat, W1, b1, W2, b2, sel, n_e):
    return pl.pallas_call(
        _dense_body,
        grid=(n_e // BE,),
        in_specs=[
            pl.BlockSpec((BE, ED), lambda i: (i, 0)),
            pl.BlockSpec((BE, U), lambda i: (i, 0)),
            pl.BlockSpec((ED, U), lambda i: (0, 0)),
            pl.BlockSpec((1, U), lambda i: (0, 0)),
            pl.BlockSpec((U, U * U), lambda i: (0, 0)),
            pl.BlockSpec((1, U * U), lambda i: (0, 0)),
            pl.BlockSpec((U * U, U), lambda i: (0, 0)),
        ],
        out_specs=pl.BlockSpec((BE, U), lambda i: (i, 0)),
        out_shape=jax.ShapeDtypeStruct((n_e, U), jnp.float32),
    )(ef_p, src_feat, W1, b1, W2, b2, sel)


def _scatter(msg1, msg2, idx1, idx2, zeros32, zeros16, ones16):
    half_rows = NROWS // 2

    @functools.partial(
        pl.kernel,
        out_type=[
            jax.ShapeDtypeStruct((2, NPAD, U), jnp.float32),
            jax.ShapeDtypeStruct((2, NPAD, 16), jnp.float32),
        ],
        mesh=_sc_mesh(),
        compiler_params=SC_PARAMS,
        scratch_types=[
            pltpu.VMEM_SHARED((NPAD, U), jnp.float32),
            pltpu.VMEM_SHARED((NPAD, 16), jnp.float32),
            pltpu.VMEM((CHUNK, 16), jnp.float32),
        ],
    )
    def scatter_kernel(msg1_hbm, msg2_hbm, idx1_hbm, idx2_hbm,
                       z32_hbm, z16_hbm, ones_hbm,
                       sums_out, cnt_out, sums_sh, cnt_sh, ones_v):
        cid = lax.axis_index("core")
        sid = lax.axis_index("subcore")
        base = sid * STRIPE
        pltpu.sync_copy(z32_hbm, sums_sh.at[pl.ds(base, STRIPE)])
        pltpu.sync_copy(z16_hbm, cnt_sh.at[pl.ds(base, STRIPE)])
        pltpu.sync_copy(ones_hbm, ones_v)
        plsc.subcore_barrier()

        def body(m_vmem, i_vmem):
            pltpu.sync_copy(m_vmem, sums_sh.at[i_vmem.at[0]], add=True)
            pltpu.sync_copy(ones_v, cnt_sh.at[i_vmem.at[0]], add=True)

        for msg_hbm, idx_hbm in ((msg1_hbm, idx1_hbm), (msg2_hbm, idx2_hbm)):
            pltpu.emit_pipeline(
                body,
                grid=(half_rows,),
                in_specs=[
                    pl.BlockSpec((CHUNK, U), lambda i: (i, 0)),
                    pl.BlockSpec((1, CHUNK), lambda i: (0, i)),
                ],
                out_specs=[],
                core_axis_name=("core", "subcore"),
                dimension_semantics=(pltpu.PARALLEL,),
            )(msg_hbm, idx_hbm)
        plsc.subcore_barrier()
        pltpu.sync_copy(sums_sh.at[pl.ds(base, STRIPE)],
                        sums_out.at[cid, pl.ds(base, STRIPE)])
        pltpu.sync_copy(cnt_sh.at[pl.ds(base, STRIPE)],
                        cnt_out.at[cid, pl.ds(base, STRIPE)])

    return scatter_kernel(msg1, msg2, idx1, idx2, zeros32, zeros16, ones16)


def _gru_body(sums_ref, cnt_ref, kz_ref, kr_ref, kh_ref,
              rz_ref, rr_ref, rh_ref, bxz_ref, bxr_ref, bxh_ref, bhr_ref,
              h0_ref, out_ref, xz_ref, xr_ref, xh_ref):
    sums = sums_ref[0, 0:N_NODES, :] + sums_ref[1, 0:N_NODES, :]
    cnt = cnt_ref[0, 0:N_NODES, 0:1] + cnt_ref[1, 0:N_NODES, 0:1]
    agg = sums / jnp.maximum(cnt, 1.0)
    xz_ref[...] = jnp.dot(agg, kz_ref[...]) + bxz_ref[...]
    xr_ref[...] = jnp.dot(agg, kr_ref[...]) + bxr_ref[...]
    xh_ref[...] = jnp.dot(agg, kh_ref[...]) + bxh_ref[...]

    bhr = bhr_ref[...]
    rz16 = rz_ref[...]
    rr16 = rr_ref[...]
    rh16 = rh_ref[...]

    def step(t, h):
        xz = xz_ref[pl.ds(t, 1), :]
        xr = xr_ref[pl.ds(t, 1), :]
        xh = xh_ref[pl.ds(t, 1), :]
        hb = h.astype(jnp.bfloat16)
        dz = jnp.dot(hb, rz16, preferred_element_type=jnp.float32)
        dr = jnp.dot(hb, rr16, preferred_element_type=jnp.float32)
        dh = jnp.dot(hb, rh16, preferred_element_type=jnp.float32)
        tz = jnp.tanh(xz + dz)
        tr = jnp.tanh(xr + dr)
        hh = jnp.tanh(xh + (tr + 1.0) * (dh + bhr))
        return 0.5 * (h + hh + tz * (h - hh))

    def block(b, h):
        for k in range(UNROLL):
            h = step(b * UNROLL + k, h)
        return h

    out_ref[...] = lax.fori_loop(0, N_NODES // UNROLL, block, h0_ref[...])


def _gru(sums_p, cnt_p, gru_kernel, gru_rkernel, gru_bias, hidden_state):
    kz = gru_kernel[:, 0:U] * 0.5
    kr = gru_kernel[:, U:2 * U] * 0.5
    kh = gru_kernel[:, 2 * U:3 * U]
    rz = (gru_rkernel[:, 0:U] * 0.5).astype(jnp.bfloat16)
    rr = (gru_rkernel[:, U:2 * U] * 0.5).astype(jnp.bfloat16)
    rh = (gru_rkernel[:, 2 * U:3 * U] * 0.5).astype(jnp.bfloat16)
    bxz = ((gru_bias[0, 0:U] + gru_bias[1, 0:U]) * 0.5).reshape(1, U)
    bxr = ((gru_bias[0, U:2 * U] + gru_bias[1, U:2 * U]) * 0.5).reshape(1, U)
    bxh = gru_bias[0, 2 * U:3 * U].reshape(1, U)
    bhr = (gru_bias[1, 2 * U:3 * U] * 0.5).reshape(1, U)

    full = lambda s: pl.BlockSpec(s, lambda i: tuple(0 for _ in s))
    return pl.pallas_call(
        _gru_body,
        grid=(1,),
        in_specs=[
            full((2, NPAD, U)),
            full((2, NPAD, 16)),
            full((U, U)), full((U, U)), full((U, U)),
            full((U, U)), full((U, U)), full((U, U)),
            full((1, U)), full((1, U)), full((1, U)), full((1, U)),
            full((1, U)),
        ],
        out_specs=pl.BlockSpec((1, U), lambda i: (0, 0)),
        out_shape=jax.ShapeDtypeStruct((1, U), jnp.float32),
        scratch_shapes=[pltpu.VMEM((N_NODES, U), jnp.float32)] * 3,
    )(sums_p, cnt_p, kz, kr, kh, rz, rr, rh, bxz, bxr, bxh, bhr, hidden_state)


def kernel(node_features, edge_features, edge_indices, hidden_state,
           W1, b1, W2, b2, gru_kernel, gru_rkernel, gru_bias):
    pad = EP - N_EDGES
    half = EP // 2
    src_idx = jnp.concatenate(
        [edge_indices[0], jnp.zeros((pad,), jnp.int32)]).reshape(1, EP)
    dst_idx = jnp.concatenate(
        [edge_indices[1], jnp.full((pad,), DUMMY, jnp.int32)]).reshape(1, EP)
    ef_p = jnp.concatenate(
        [edge_features, jnp.zeros((pad, ED), jnp.float32)], axis=0)

    sel = jnp.asarray(np.repeat(np.eye(U, dtype=np.float32), U, axis=0))
    zeros32 = jnp.zeros((STRIPE, U), jnp.float32)
    zeros16 = jnp.zeros((STRIPE, 16), jnp.float32)
    ones16 = jnp.ones((CHUNK, 16), jnp.float32)
    b1r = b1.reshape(1, U)
    b2r = b2.reshape(1, U * U)

    g1 = _gather(node_features, src_idx[:, :half], NROWS // 2)
    g2 = _gather(node_features, src_idx[:, half:], NROWS // 2)
    d1 = _dense(ef_p[:half], g1, W1, b1r, W2, b2r, sel, half)
    d2 = _dense(ef_p[half:], g2, W1, b1r, W2, b2r, sel, half)
    sums_p, cnt_p = _scatter(d1, d2, dst_idx[:, :half], dst_idx[:, half:],
                             zeros32, zeros16, ones16)
    new_state = _gru(sums_p, cnt_p, gru_kernel, gru_rkernel, gru_bias,
                     hidden_state)
    return new_state[0], new_state
